# Initial kernel scaffold; baseline (speedup 1.0000x reference)
import jax
import jax.numpy as jnp
from jax.experimental import pallas as pl


def kernel(pred_ids, gt_ids, table):
    raise NotImplementedError("write your pallas kernel here")


# speedup vs baseline: 6.4172x; 6.4172x over previous
import functools

import jax
import jax.numpy as jnp
from jax import lax
from jax.experimental import pallas as pl
from jax.experimental.pallas import tpu as pltpu
from jax.experimental.pallas import tpu_sc as plsc

_V = 10000
_D = 256
_B = 16384
_S = 200

_NC, _NS, _L = 2, 16, 16
_NW = _NC * _NS
_RPW = _B // _NW
_G = 16
_NG = _RPW // _G
_CH = 2
_SC = _S // _CH
_DV = _D // _L


def _rsqrt_nr(x):
    xi = plsc.bitcast(x, jnp.int32)
    yi = jnp.int32(0x5F3759DF) - (xi >> 1)
    y = plsc.bitcast(yi, jnp.float32)
    for _ in range(3):
        y = y * (1.5 - 0.5 * x * y * y)
    return y


def _sc_body(pred_hbm, gt_hbm, table_hbm, out_hbm,
             idx_p, idx_g, buf_p, buf_g, sums_p, sums_g, rewards, sem):
    wid = lax.axis_index("s") * _NC + lax.axis_index("c")
    base = wid * _RPW
    zero = jnp.zeros((_L,), jnp.float32)
    rows16 = lax.iota(jnp.int32, _L) * _D

    def reduce_side(buf):
        def tok(j, accs):
            return tuple(accs[k] + buf[j, pl.ds(k * _L, _L)]
                         for k in range(_DV))
        return lax.fori_loop(0, _S, tok, (zero,) * _DV)

    def group_body(g, carry):
        rbase = base + g * _G
        pltpu.sync_copy(pred_hbm.at[pl.ds(rbase, _G)], idx_p)
        pltpu.sync_copy(gt_hbm.at[pl.ds(rbase, _G)], idx_g)

        def row_body(i, c2):
            cps = []
            for c in range(_CH):
                cps.append(pltpu.async_copy(
                    table_hbm.at[idx_p.at[i, c]],
                    buf_p.at[pl.ds(c * _SC, _SC)], sem))
                cps.append(pltpu.async_copy(
                    table_hbm.at[idx_g.at[i, c]],
                    buf_g.at[pl.ds(c * _SC, _SC)], sem))
            for cp in cps:
                cp.wait()
            accp = reduce_side(buf_p)
            accg = reduce_side(buf_g)
            for k in range(_DV):
                sums_p[pl.ds(i * _D + k * _L, _L)] = accp[k]
                sums_g[pl.ds(i * _D + k * _L, _L)] = accg[k]
            return c2

        lax.fori_loop(0, _G, row_body, 0)

        def fin(d, carry3):
            dot, np_, ng_ = carry3
            idxv = rows16 + d
            p = plsc.load_gather(sums_p, [idxv])
            q = plsc.load_gather(sums_g, [idxv])
            return dot + p * q, np_ + p * p, ng_ + q * q

        dot, np_, ng_ = lax.fori_loop(0, _D, fin, (zero, zero, zero))
        inv2 = jnp.float32(1.0 / (_S * _S))
        np_m = jnp.maximum(np_ * inv2, 1e-16)
        ng_m = jnp.maximum(ng_ * inv2, 1e-16)
        sim = dot * inv2 * _rsqrt_nr(np_m * ng_m)
        rewards[pl.ds(g * _G, _G)] = 0.7 * jnp.maximum(sim, 0.0)
        return carry

    lax.fori_loop(0, _NG, group_body, 0)
    pltpu.sync_copy(rewards, out_hbm.at[pl.ds(base, _RPW)])


def _make_sc_kernel(interpret=False):
    mesh = plsc.VectorSubcoreMesh(core_axis_name="c", subcore_axis_name="s",
                                  num_cores=_NC, num_subcores=_NS)
    return pl.kernel(
        _sc_body,
        out_type=jax.ShapeDtypeStruct((_B,), jnp.float32),
        mesh=mesh,
        scratch_types=[
            pltpu.VMEM((_G, _CH, _SC), jnp.int32),
            pltpu.VMEM((_G, _CH, _SC), jnp.int32),
            pltpu.VMEM((_S, _D), jnp.float32),
            pltpu.VMEM((_S, _D), jnp.float32),
            pltpu.VMEM((_G * _D,), jnp.float32),
            pltpu.VMEM((_G * _D,), jnp.float32),
            pltpu.VMEM((_RPW,), jnp.float32),
            pltpu.SemaphoreType.DMA,
        ],
        compiler_params=pltpu.CompilerParams(use_tc_tiling_on_sc=False,
                                             needs_layout_passes=False),
        interpret=interpret,
    )


@jax.jit
def kernel(pred_ids, gt_ids, table):
    pred3 = pred_ids.astype(jnp.int32).reshape(_B, _CH, _SC)
    gt3 = gt_ids.astype(jnp.int32).reshape(_B, _CH, _SC)
    return _make_sc_kernel()(pred3, gt3, table)


# speedup vs baseline: 15.3454x; 2.3913x over previous
import functools

import jax
import jax.numpy as jnp
from jax import lax
from jax.experimental import pallas as pl
from jax.experimental.pallas import tpu as pltpu
from jax.experimental.pallas import tpu_sc as plsc

_V = 10000
_D = 256
_B = 16384
_S = 200

_NC, _NS, _L = 2, 16, 16
_NW = _NC * _NS
_RPW = _B // _NW
_G = 16
_NG = _RPW // _G
_CH = 2
_SC = _S // _CH
_DV = _D // _L
_PK = _D // (2 * _L)


def _rsqrt_nr(x):
    xi = plsc.bitcast(x, jnp.int32)
    yi = jnp.int32(0x5F3759DF) - (xi >> 1)
    y = plsc.bitcast(yi, jnp.float32)
    for _ in range(3):
        y = y * (1.5 - 0.5 * x * y * y)
    return y


def _sc_body(pred_hbm, gt_hbm, table_hbm, out_hbm,
             idx_p, idx_g, bufs, sums_p, sums_g, rewards, sem0, sem1):
    wid = lax.axis_index("s") * _NC + lax.axis_index("c")
    base = wid * _RPW
    zero = jnp.zeros((_L,), jnp.float32)
    rows16 = lax.iota(jnp.int32, _L) * _D
    sems = (sem0, sem1)

    def row_copies(i, par):
        cps = []
        for side, idx in ((0, idx_p), (1, idx_g)):
            for c in range(_CH):
                cps.append(pltpu.make_async_copy(
                    table_hbm.at[idx.at[i, c]],
                    bufs.at[par, side, pl.ds(c * _SC, _SC)], sems[par]))
        return cps

    def issue_row(i, par):
        for cp in row_copies(i, par):
            cp.start()

    def wait_row(i, par):
        for cp in row_copies(i, par):
            cp.wait()

    def reduce_row(i, par):
        def tok(j, accs):
            out = []
            for side in range(2):
                a = list(accs[side * _DV:(side + 1) * _DV])
                for k in range(_PK):
                    v = bufs[par, side, j, pl.ds(k * 2 * _L, 2 * _L)]
                    lo, hi = plsc.unpack(v, format=plsc.PackFormat.INTERLEAVED)
                    a[2 * k] = a[2 * k] + lo
                    a[2 * k + 1] = a[2 * k + 1] + hi
                out.extend(a)
            return tuple(out)

        accs = lax.fori_loop(0, _S, tok, (zero,) * (2 * _DV))
        for k in range(_DV):
            sums_p[pl.ds(i * _D + k * _L, _L)] = accs[k]
            sums_g[pl.ds(i * _D + k * _L, _L)] = accs[_DV + k]

    def group_body(g, carry):
        rbase = base + g * _G
        pltpu.sync_copy(pred_hbm.at[pl.ds(rbase, _G)], idx_p)
        pltpu.sync_copy(gt_hbm.at[pl.ds(rbase, _G)], idx_g)
        issue_row(0, 0)

        def pair_body(ii, c2):
            a = 2 * ii
            b = a + 1
            issue_row(b, 1)
            wait_row(a, 0)
            reduce_row(a, 0)

            @pl.when(b + 1 < _G)
            def _():
                issue_row(b + 1, 0)

            wait_row(b, 1)
            reduce_row(b, 1)
            return c2

        lax.fori_loop(0, _G // 2, pair_body, 0)

        def fin(d, carry3):
            dot, np_, ng_ = carry3
            idxv = rows16 + d
            p = plsc.load_gather(sums_p, [idxv])
            q = plsc.load_gather(sums_g, [idxv])
            return dot + p * q, np_ + p * p, ng_ + q * q

        dot, np_, ng_ = lax.fori_loop(0, _D, fin, (zero, zero, zero))
        inv2 = jnp.float32(1.0 / (_S * _S))
        np_m = jnp.maximum(np_ * inv2, 1e-16)
        ng_m = jnp.maximum(ng_ * inv2, 1e-16)
        sim = dot * inv2 * _rsqrt_nr(np_m * ng_m)
        rewards[pl.ds(g * _G, _G)] = 0.7 * jnp.maximum(sim, 0.0)
        return carry

    lax.fori_loop(0, _NG, group_body, 0)
    pltpu.sync_copy(rewards, out_hbm.at[pl.ds(base, _RPW)])


def _make_sc_kernel(interpret=False):
    mesh = plsc.VectorSubcoreMesh(core_axis_name="c", subcore_axis_name="s",
                                  num_cores=_NC, num_subcores=_NS)
    return pl.kernel(
        _sc_body,
        out_type=jax.ShapeDtypeStruct((_B,), jnp.float32),
        mesh=mesh,
        scratch_types=[
            pltpu.VMEM((_G, _CH, _SC), jnp.int32),
            pltpu.VMEM((_G, _CH, _SC), jnp.int32),
            pltpu.VMEM((2, 2, _S, _D), jnp.bfloat16),
            pltpu.VMEM((_G * _D,), jnp.float32),
            pltpu.VMEM((_G * _D,), jnp.float32),
            pltpu.VMEM((_RPW,), jnp.float32),
            pltpu.SemaphoreType.DMA,
            pltpu.SemaphoreType.DMA,
        ],
        compiler_params=pltpu.CompilerParams(use_tc_tiling_on_sc=False,
                                             needs_layout_passes=False),
        interpret=interpret,
    )


@jax.jit
def kernel(pred_ids, gt_ids, table):
    pred3 = pred_ids.astype(jnp.int32).reshape(_B, _CH, _SC)
    gt3 = gt_ids.astype(jnp.int32).reshape(_B, _CH, _SC)
    table_bf = table.astype(jnp.bfloat16)
    return _make_sc_kernel()(pred3, gt3, table_bf)


# speedup vs baseline: 16.2062x; 1.0561x over previous
import functools

import jax
import jax.numpy as jnp
from jax import lax
from jax.experimental import pallas as pl
from jax.experimental.pallas import tpu as pltpu
from jax.experimental.pallas import tpu_sc as plsc

_V = 10000
_D = 256
_B = 16384
_S = 200

_NC, _NS, _L = 2, 16, 16
_NW = _NC * _NS
_RPW = _B // _NW
_G = 16
_NG = _RPW // _G
_CH = 2
_SC = _S // _CH
_DV = _D // _L
_PK = _D // (2 * _L)
_TCH = 4


def _rsqrt_nr(x):
    xi = plsc.bitcast(x, jnp.int32)
    yi = jnp.int32(0x5F3759DF) - (xi >> 1)
    y = plsc.bitcast(yi, jnp.float32)
    for _ in range(3):
        y = y * (1.5 - 0.5 * x * y * y)
    return y


def _sc_body(pred_hbm, gt_hbm, table_hbm, out_hbm,
             idx_p, idx_g, bufs, sums_p, sums_g, rewards, sem0, sem1):
    wid = lax.axis_index("s") * _NC + lax.axis_index("c")
    base = wid * _RPW
    zero = jnp.zeros((_L,), jnp.float32)
    rows16 = lax.iota(jnp.int32, _L) * _D
    sems = (sem0, sem1)

    def row_copies(i, par):
        cps = []
        for side, idx in ((0, idx_p), (1, idx_g)):
            for c in range(_CH):
                cps.append(pltpu.make_async_copy(
                    table_hbm.at[idx.at[i, c]],
                    bufs.at[par, side, pl.ds(c * _SC, _SC)], sems[par]))
        return cps

    def issue_row(i, par):
        for cp in row_copies(i, par):
            cp.start()

    def wait_row(i, par):
        for cp in row_copies(i, par):
            cp.wait()

    def reduce_row(i, par):
        zero_b = jnp.zeros((2 * _L,), jnp.bfloat16)

        def chunk(jj, accs):
            f = list(accs)
            j0 = jj * _TCH
            for side in range(2):
                for k in range(_PK):
                    b = bufs[par, side, j0, pl.ds(k * 2 * _L, 2 * _L)]
                    for t in range(1, _TCH):
                        b = b + bufs[par, side, j0 + t, pl.ds(k * 2 * _L, 2 * _L)]
                    lo, hi = plsc.unpack(b, format=plsc.PackFormat.INTERLEAVED)
                    f[side * _DV + 2 * k] += lo
                    f[side * _DV + 2 * k + 1] += hi
            return tuple(f)

        accs = lax.fori_loop(0, _S // _TCH, chunk, (zero,) * (2 * _DV))
        for k in range(_DV):
            sums_p[pl.ds(i * _D + k * _L, _L)] = accs[k]
            sums_g[pl.ds(i * _D + k * _L, _L)] = accs[_DV + k]

    def group_body(g, carry):
        rbase = base + g * _G
        pltpu.sync_copy(pred_hbm.at[pl.ds(rbase, _G)], idx_p)
        pltpu.sync_copy(gt_hbm.at[pl.ds(rbase, _G)], idx_g)
        issue_row(0, 0)

        def pair_body(ii, c2):
            a = 2 * ii
            b = a + 1
            issue_row(b, 1)
            wait_row(a, 0)
            reduce_row(a, 0)

            @pl.when(b + 1 < _G)
            def _():
                issue_row(b + 1, 0)

            wait_row(b, 1)
            reduce_row(b, 1)
            return c2

        lax.fori_loop(0, _G // 2, pair_body, 0)

        def fin(d, carry3):
            dot, np_, ng_ = carry3
            idxv = rows16 + d
            p = plsc.load_gather(sums_p, [idxv])
            q = plsc.load_gather(sums_g, [idxv])
            return dot + p * q, np_ + p * p, ng_ + q * q

        dot, np_, ng_ = lax.fori_loop(0, _D, fin, (zero, zero, zero))
        inv2 = jnp.float32(1.0 / (_S * _S))
        np_m = jnp.maximum(np_ * inv2, 1e-16)
        ng_m = jnp.maximum(ng_ * inv2, 1e-16)
        sim = dot * inv2 * _rsqrt_nr(np_m * ng_m)
        rewards[pl.ds(g * _G, _G)] = 0.7 * jnp.maximum(sim, 0.0)
        return carry

    lax.fori_loop(0, _NG, group_body, 0)
    pltpu.sync_copy(rewards, out_hbm.at[pl.ds(base, _RPW)])


def _make_sc_kernel(interpret=False):
    mesh = plsc.VectorSubcoreMesh(core_axis_name="c", subcore_axis_name="s",
                                  num_cores=_NC, num_subcores=_NS)
    return pl.kernel(
        _sc_body,
        out_type=jax.ShapeDtypeStruct((_B,), jnp.float32),
        mesh=mesh,
        scratch_types=[
            pltpu.VMEM((_G, _CH, _SC), jnp.int32),
            pltpu.VMEM((_G, _CH, _SC), jnp.int32),
            pltpu.VMEM((2, 2, _S, _D), jnp.bfloat16),
            pltpu.VMEM((_G * _D,), jnp.float32),
            pltpu.VMEM((_G * _D,), jnp.float32),
            pltpu.VMEM((_RPW,), jnp.float32),
            pltpu.SemaphoreType.DMA,
            pltpu.SemaphoreType.DMA,
        ],
        compiler_params=pltpu.CompilerParams(use_tc_tiling_on_sc=False,
                                             needs_layout_passes=False),
        interpret=interpret,
    )


@jax.jit
def kernel(pred_ids, gt_ids, table):
    pred3 = pred_ids.astype(jnp.int32).reshape(_B, _CH, _SC)
    gt3 = gt_ids.astype(jnp.int32).reshape(_B, _CH, _SC)
    table_bf = table.astype(jnp.bfloat16)
    return _make_sc_kernel()(pred3, gt3, table_bf)
